# Initial kernel scaffold; baseline (speedup 1.0000x reference)
import jax
import jax.numpy as jnp
from jax.experimental import pallas as pl


def kernel(x, edge_index, edge_weight, W0, b0, W1, b1, W2, b2, W3, b3):
    raise NotImplementedError("write your pallas kernel here")


# speedup vs baseline: 6.7361x; 6.7361x over previous
import functools

import jax
import jax.numpy as jnp
from jax import lax
from jax.experimental import pallas as pl
from jax.experimental.pallas import tpu as pltpu
from jax.experimental.pallas import tpu_sc as plsc

N_NODES = 10000
N_EDGES = 320000
NC = 2
NS = 16
NW = NC * NS
E_PAD = 327680
BLOCK = 1024
HALF = 512
ROW_BLK = 1000
SEG = 624
OUT_SLICE = 208

_GDN = lax.GatherDimensionNumbers(
    offset_dims=(), collapsed_slice_dims=(0,), start_index_map=(0,))


def _bcast_lane(v16, m):
    idx = jnp.full((16, 1), m, jnp.int32)
    return lax.gather(v16, idx, _GDN, (1,),
                      mode=lax.GatherScatterMode.PROMISE_IN_BOUNDS)


def _make_sc_msg(f_pad, fsplit):
    nq = f_pad // 16
    per_tile = E_PAD // NS if fsplit else E_PAD // NW
    n_blocks = per_tile // BLOCK
    s_rows = NC * N_NODES if fsplit else N_NODES
    mesh = plsc.VectorSubcoreMesh(core_axis_name="c", subcore_axis_name="s")

    @functools.partial(
        pl.kernel,
        out_type=jax.ShapeDtypeStruct((NC, N_NODES, f_pad), jnp.float32),
        mesh=mesh,
        scratch_types=[
            pltpu.VMEM((BLOCK // 128, 128), jnp.int32),
            pltpu.VMEM((BLOCK // 128, 128), jnp.int32),
            pltpu.VMEM((BLOCK // 16, 16), jnp.float32),
            pltpu.VMEM((HALF, f_pad), jnp.float32),
            pltpu.VMEM((OUT_SLICE, f_pad), jnp.float32),
            pltpu.VMEM_SHARED((N_NODES, f_pad), jnp.float32),
            pltpu.SemaphoreType.DMA,
        ],
        compiler_params=pltpu.CompilerParams(use_tc_tiling_on_sc=False),
    )
    def body(s_hbm, row_hbm, col_hbm, w_hbm, out_hbm,
             idxr, idxc, wv, rows, stage, acc, sem):
        cid = lax.axis_index("c")
        sid = lax.axis_index("s")
        wid = sid if fsplit else sid * NC + cid

        def zero_row(i, carry):
            for q in range(nq):
                stage[i, pl.ds(q * 16, 16)] = jnp.zeros((16,), jnp.float32)
            return carry
        lax.fori_loop(0, OUT_SLICE, zero_row, 0)
        for k in range(SEG // OUT_SLICE):
            pltpu.sync_copy(stage,
                            acc.at[pl.ds(sid * SEG + k * OUT_SLICE, OUT_SLICE)])
        @pl.when(sid == NS - 1)
        def _():
            pltpu.sync_copy(stage.at[pl.ds(0, 16)],
                            acc.at[pl.ds(NS * SEG, N_NODES - NS * SEG)])
        plsc.subcore_barrier()

        def block(g, carry):
            r0 = wid * (per_tile // 128) + g * (BLOCK // 128)
            w0 = wid * (per_tile // 16) + g * (BLOCK // 16)
            pltpu.sync_copy(row_hbm.at[pl.ds(r0, BLOCK // 128)], idxr)
            pltpu.sync_copy(col_hbm.at[pl.ds(r0, BLOCK // 128)], idxc)
            pltpu.sync_copy(w_hbm.at[pl.ds(w0, BLOCK // 16)], wv)
            if fsplit:
                off = jnp.full((16,), cid * N_NODES, jnp.int32)
                def addoff(i, c2):
                    for qq in range(128 // 16):
                        idxr[i, pl.ds(qq * 16, 16)] = (
                            idxr[i, pl.ds(qq * 16, 16)] + off)
                    return c2
                lax.fori_loop(0, BLOCK // 128, addoff, 0)
            for h in range(BLOCK // HALF):
                cps = [pltpu.async_copy(
                           s_hbm.at[idxr.at[h * (HALF // 128) + j]],
                           rows.at[pl.ds(j * 128, 128)], sem)
                       for j in range(HALF // 128)]
                for c in cps:
                    c.wait()

                def scale(l, carry2):
                    w16 = wv[h * (HALF // 16) + l]
                    e0 = l * 16
                    for m in range(16):
                        wb = _bcast_lane(w16, m)
                        for q in range(nq):
                            rows[e0 + m, pl.ds(q * 16, 16)] = (
                                rows[e0 + m, pl.ds(q * 16, 16)] * wb)
                    return carry2
                lax.fori_loop(0, HALF // 16, scale, 0)

                for j in range(HALF // 128):
                    pltpu.sync_copy(rows.at[pl.ds(j * 128, 128)],
                                    acc.at[idxc.at[h * (HALF // 128) + j]],
                                    add=True)
            return carry
        lax.fori_loop(0, n_blocks, block, 0)
        plsc.subcore_barrier()

        for k in range(SEG // OUT_SLICE):
            off = sid * SEG + k * OUT_SLICE
            pltpu.sync_copy(acc.at[pl.ds(off, OUT_SLICE)], stage)
            pltpu.sync_copy(stage, out_hbm.at[cid, pl.ds(off, OUT_SLICE)])
        @pl.when(sid == NS - 1)
        def _():
            tail = N_NODES - NS * SEG
            pltpu.sync_copy(acc.at[pl.ds(NS * SEG, tail)],
                            stage.at[pl.ds(0, tail)])
            pltpu.sync_copy(stage.at[pl.ds(0, tail)],
                            out_hbm.at[cid, pl.ds(NS * SEG, tail)])

    return body


_sc_msg64_fs = _make_sc_msg(64, fsplit=True)
_sc_msg64_es = _make_sc_msg(64, fsplit=False)
_sc_msg16_es = _make_sc_msg(16, fsplit=False)


def _tc_dinv(degp):
    def body(p_ref, o_ref):
        d = p_ref[0, :, 0:1] + p_ref[1, :, 0:1] + 1.0
        o_ref[...] = lax.rsqrt(d)
    return pl.pallas_call(
        body,
        out_shape=jax.ShapeDtypeStruct((N_NODES, 1), jnp.float32),
    )(degp)


def _tc_first(x, w0p, dinv):
    def body(x_ref, w_ref, di_ref, o_ref):
        s = jnp.dot(x_ref[...], w_ref[...],
                    preferred_element_type=jnp.float32) * di_ref[...]
        o_ref[0] = s[:, :64]
        o_ref[1] = s[:, 64:]
    return pl.pallas_call(
        body,
        grid=(N_NODES // ROW_BLK,),
        in_specs=[
            pl.BlockSpec((ROW_BLK, 128), lambda i: (i, 0)),
            pl.BlockSpec((128, 128), lambda i: (0, 0)),
            pl.BlockSpec((ROW_BLK, 1), lambda i: (i, 0)),
        ],
        out_specs=pl.BlockSpec((2, ROW_BLK, 64), lambda i: (0, i, 0)),
        out_shape=jax.ShapeDtypeStruct((2, N_NODES, 64), jnp.float32),
    )(x, w0p, dinv)


def _tc_mid_fs(p, s2, dinv, bp, wp, fp_out, out_split):
    def body(p_ref, s_ref, di_ref, b_ref, w_ref, o_ref):
        di = di_ref[...]
        pcat = jnp.concatenate([p_ref[0], p_ref[1]], axis=1)
        scat = jnp.concatenate([s_ref[0], s_ref[1]], axis=1)
        h = jnp.maximum(di * pcat + scat * di + b_ref[...], 0.0)
        s_new = jnp.dot(h, w_ref[...],
                        preferred_element_type=jnp.float32) * di
        if out_split:
            o_ref[0] = s_new[:, :64]
            o_ref[1] = s_new[:, 64:]
        else:
            o_ref[...] = s_new
    if out_split:
        out_spec = pl.BlockSpec((2, ROW_BLK, 64), lambda i: (0, i, 0))
        out_shape = jax.ShapeDtypeStruct((2, N_NODES, 64), jnp.float32)
    else:
        out_spec = pl.BlockSpec((ROW_BLK, fp_out), lambda i: (i, 0))
        out_shape = jax.ShapeDtypeStruct((N_NODES, fp_out), jnp.float32)
    return pl.pallas_call(
        body,
        grid=(N_NODES // ROW_BLK,),
        in_specs=[
            pl.BlockSpec((2, ROW_BLK, 64), lambda i: (0, i, 0)),
            pl.BlockSpec((2, ROW_BLK, 64), lambda i: (0, i, 0)),
            pl.BlockSpec((ROW_BLK, 1), lambda i: (i, 0)),
            pl.BlockSpec((1, 128), lambda i: (0, 0)),
            pl.BlockSpec((128, fp_out), lambda i: (0, 0)),
        ],
        out_specs=out_spec,
        out_shape=out_shape,
    )(p, s2, dinv, bp, wp)


def _tc_mid_es(p, s_prev, dinv, bp, wp, fp_in, fp_out):
    def body(p_ref, s_ref, di_ref, b_ref, w_ref, o_ref):
        di = di_ref[...]
        h = jnp.maximum(di * (p_ref[0] + p_ref[1]) + s_ref[...] * di
                        + b_ref[...], 0.0)
        o_ref[...] = jnp.dot(h, w_ref[...],
                             preferred_element_type=jnp.float32) * di
    return pl.pallas_call(
        body,
        grid=(N_NODES // ROW_BLK,),
        in_specs=[
            pl.BlockSpec((2, ROW_BLK, fp_in), lambda i: (0, i, 0)),
            pl.BlockSpec((ROW_BLK, fp_in), lambda i: (i, 0)),
            pl.BlockSpec((ROW_BLK, 1), lambda i: (i, 0)),
            pl.BlockSpec((1, fp_in), lambda i: (0, 0)),
            pl.BlockSpec((fp_in, fp_out), lambda i: (0, 0)),
        ],
        out_specs=pl.BlockSpec((ROW_BLK, fp_out), lambda i: (i, 0)),
        out_shape=jax.ShapeDtypeStruct((N_NODES, fp_out), jnp.float32),
    )(p, s_prev, dinv, bp, wp)


def _tc_last(p, s_prev, dinv, bp):
    def body(p_ref, s_ref, di_ref, b_ref, o_ref):
        di = di_ref[...]
        o_ref[...] = di * (p_ref[0] + p_ref[1]) + s_ref[...] * di + b_ref[...]
    return pl.pallas_call(
        body,
        grid=(N_NODES // ROW_BLK,),
        in_specs=[
            pl.BlockSpec((2, ROW_BLK, 16), lambda i: (0, i, 0)),
            pl.BlockSpec((ROW_BLK, 16), lambda i: (i, 0)),
            pl.BlockSpec((ROW_BLK, 1), lambda i: (i, 0)),
            pl.BlockSpec((1, 16), lambda i: (0, 0)),
        ],
        out_specs=pl.BlockSpec((ROW_BLK, 16), lambda i: (i, 0)),
        out_shape=jax.ShapeDtypeStruct((N_NODES, 16), jnp.float32),
    )(p, s_prev, dinv, bp)


def _pad2(w, r, c):
    return jnp.pad(w, ((0, r - w.shape[0]), (0, c - w.shape[1])))


def kernel(x, edge_index, edge_weight, W0, b0, W1, b1, W2, b2, W3, b3):
    row = edge_index[0].astype(jnp.int32)
    col = edge_index[1].astype(jnp.int32)
    padn = E_PAD - N_EDGES
    rowp = jnp.concatenate([row, jnp.zeros((padn,), jnp.int32)]
                           ).reshape(E_PAD // 128, 128)
    colp = jnp.concatenate([col, jnp.zeros((padn,), jnp.int32)]
                           ).reshape(E_PAD // 128, 128)
    wp = jnp.concatenate([edge_weight, jnp.zeros((padn,), jnp.float32)]
                         ).reshape(E_PAD // 16, 16)

    W0p = _pad2(W0, 128, 128)
    W1p = _pad2(W1, 128, 128)
    W2p = _pad2(W2, 128, 64)
    W3p = _pad2(W3, 64, 16)
    b0p = jnp.pad(b0, (0, 28)).reshape(1, 128)
    b1p = jnp.pad(b1, (0, 28)).reshape(1, 128)
    b2p = jnp.pad(b2, (0, 14)).reshape(1, 64)
    b3p = jnp.pad(b3, (0, 10)).reshape(1, 16)

    ones16 = jnp.ones((N_NODES, 16), jnp.float32)
    degp = _sc_msg16_es(ones16, rowp, colp, wp)
    dinv = _tc_dinv(degp)

    s0 = _tc_first(x, W0p, dinv)
    p0 = _sc_msg64_fs(s0.reshape(2 * N_NODES, 64), rowp, colp, wp)
    s1 = _tc_mid_fs(p0, s0, dinv, b0p, W1p, 128, out_split=True)
    p1 = _sc_msg64_fs(s1.reshape(2 * N_NODES, 64), rowp, colp, wp)
    s2 = _tc_mid_fs(p1, s1, dinv, b1p, W2p, 64, out_split=False)
    p2 = _sc_msg64_es(s2, rowp, colp, wp)
    s3 = _tc_mid_es(p2, s2, dinv, b2p, W3p, 64, 16)
    p3 = _sc_msg16_es(s3, rowp, colp, wp)
    out = _tc_last(p3, s3, dinv, b3p)
    return out[:, :6]


# speedup vs baseline: 9.2088x; 1.3671x over previous
import functools

import jax
import jax.numpy as jnp
from jax import lax
from jax.experimental import pallas as pl
from jax.experimental.pallas import tpu as pltpu
from jax.experimental.pallas import tpu_sc as plsc

N_NODES = 10000
N_EDGES = 320000
NC = 2
NS = 16
NW = NC * NS
E_PAD = 327680
SUPER = 5120
HALF = 256
HALVES = SUPER // HALF
ROW_BLK = 1000
SEG = 624
OUT_SLICE = 208

_GDN = lax.GatherDimensionNumbers(
    offset_dims=(), collapsed_slice_dims=(0,), start_index_map=(0,))


def _bcast_lane(v16, m):
    idx = jnp.full((16, 1), m, jnp.int32)
    return lax.gather(v16, idx, _GDN, (1,),
                      mode=lax.GatherScatterMode.PROMISE_IN_BOUNDS)


def _make_sc_msg(f_pad, fsplit):
    nq = f_pad // 16
    per_tile = E_PAD // NS if fsplit else E_PAD // NW
    n_super = per_tile // SUPER
    mesh = plsc.VectorSubcoreMesh(core_axis_name="c", subcore_axis_name="s")

    @functools.partial(
        pl.kernel,
        out_type=jax.ShapeDtypeStruct((NC, N_NODES, f_pad), jnp.float32),
        mesh=mesh,
        scratch_types=[
            pltpu.VMEM((SUPER // 128, 128), jnp.int32),
            pltpu.VMEM((SUPER // 128, 128), jnp.int32),
            pltpu.VMEM((SUPER // 16, 16), jnp.float32),
            pltpu.VMEM((HALF, f_pad), jnp.float32),
            pltpu.VMEM((HALF, f_pad), jnp.float32),
            pltpu.VMEM_SHARED((N_NODES, f_pad), jnp.float32),
            pltpu.SemaphoreType.DMA,
            pltpu.SemaphoreType.DMA,
            pltpu.SemaphoreType.DMA,
            pltpu.SemaphoreType.DMA,
        ],
        compiler_params=pltpu.CompilerParams(use_tc_tiling_on_sc=False),
    )
    def body(s_hbm, row_hbm, col_hbm, w_hbm, out_hbm,
             idxr, idxc, wv, rows_a, rows_b, acc,
             sem_ga, sem_gb, sem_sa, sem_sb):
        cid = lax.axis_index("c")
        sid = lax.axis_index("s")
        wid = sid if fsplit else sid * NC + cid
        rows = (rows_a, rows_b)
        stage = rows_a.at[pl.ds(0, OUT_SLICE)]
        sem_g = (sem_ga, sem_gb)
        sem_s = (sem_sa, sem_sb)

        def zero_row(i, carry):
            for q in range(nq):
                stage[i, pl.ds(q * 16, 16)] = jnp.zeros((16,), jnp.float32)
            return carry
        lax.fori_loop(0, OUT_SLICE, zero_row, 0)
        for k in range(SEG // OUT_SLICE):
            pltpu.sync_copy(stage,
                            acc.at[pl.ds(sid * SEG + k * OUT_SLICE, OUT_SLICE)])
        @pl.when(sid == NS - 1)
        def _():
            pltpu.sync_copy(stage.at[pl.ds(0, 16)],
                            acc.at[pl.ds(NS * SEG, N_NODES - NS * SEG)])
        plsc.subcore_barrier()

        def issue_gather(h, p):
            for j in range(HALF // 128):
                pltpu.async_copy(s_hbm.at[idxr.at[h * (HALF // 128) + j]],
                                 rows[p].at[pl.ds(j * 128, 128)], sem_g[p])

        def wait_gather(p):
            for j in range(HALF // 128):
                pltpu.make_async_copy(
                    s_hbm.at[idxr.at[j]],
                    rows[p].at[pl.ds(j * 128, 128)], sem_g[p]).wait()

        def issue_scatter(h, p):
            for j in range(HALF // 128):
                pltpu.sync_copy(rows[p].at[pl.ds(j * 128, 128)],
                                acc.at[idxc.at[h * (HALF // 128) + j]],
                                add=True)

        def wait_scatter(p):
            pass

        def scale(h, p):
            def _scale_body(l, carry):
                w16 = wv[h * (HALF // 16) + l]
                e0 = l * 16
                for m in range(16):
                    wb = _bcast_lane(w16, m)
                    for q in range(nq):
                        rows[p][e0 + m, pl.ds(q * 16, 16)] = (
                            rows[p][e0 + m, pl.ds(q * 16, 16)] * wb)
                return carry
            lax.fori_loop(0, HALF // 16, _scale_body, 0)

        for sc_i in range(n_super):
            base = wid * per_tile + sc_i * SUPER
            pltpu.sync_copy(row_hbm.at[pl.ds(base // 128, SUPER // 128)], idxr)
            pltpu.sync_copy(col_hbm.at[pl.ds(base // 128, SUPER // 128)], idxc)
            pltpu.sync_copy(w_hbm.at[pl.ds(base // 16, SUPER // 16)], wv)
            if fsplit:
                off = jnp.full((16,), cid * N_NODES, jnp.int32)
                def addoff(i, c2):
                    for qq in range(128 // 16):
                        idxr[i, pl.ds(qq * 16, 16)] = (
                            idxr[i, pl.ds(qq * 16, 16)] + off)
                    return c2
                lax.fori_loop(0, SUPER // 128, addoff, 0)

            issue_gather(0, 0)
            def pair(g, carry):
                for hh in range(2):
                    p = hh
                    q = 1 - hh
                    h = g * 2 + hh
                    wait_gather(p)
                    @pl.when(h > 0)
                    def _():
                        wait_scatter(q)
                    @pl.when(h < HALVES - 1)
                    def _():
                        issue_gather(h + 1, q)
                    scale(h, p)
                    issue_scatter(h, p)
                return carry
            lax.fori_loop(0, HALVES // 2, pair, 0)
            wait_scatter(0)
            wait_scatter(1)
        plsc.subcore_barrier()

        for k in range(SEG // OUT_SLICE):
            off = sid * SEG + k * OUT_SLICE
            pltpu.sync_copy(acc.at[pl.ds(off, OUT_SLICE)], stage)
            pltpu.sync_copy(stage, out_hbm.at[cid, pl.ds(off, OUT_SLICE)])
        @pl.when(sid == NS - 1)
        def _():
            tail = N_NODES - NS * SEG
            pltpu.sync_copy(acc.at[pl.ds(NS * SEG, tail)],
                            stage.at[pl.ds(0, tail)])
            pltpu.sync_copy(stage.at[pl.ds(0, tail)],
                            out_hbm.at[cid, pl.ds(NS * SEG, tail)])

    return body


_sc_msg64_fs = _make_sc_msg(64, fsplit=True)
_sc_msg64_es = _make_sc_msg(64, fsplit=False)
_sc_msg16_es = _make_sc_msg(16, fsplit=False)


def _tc_dinv(degp):
    def body(p_ref, o_ref):
        d = p_ref[0, :, 0:1] + p_ref[1, :, 0:1] + 1.0
        o_ref[...] = lax.rsqrt(d)
    return pl.pallas_call(
        body,
        out_shape=jax.ShapeDtypeStruct((N_NODES, 1), jnp.float32),
    )(degp)


def _tc_first(x, w0p, dinv):
    def body(x_ref, w_ref, di_ref, o_ref):
        s = jnp.dot(x_ref[...], w_ref[...],
                    preferred_element_type=jnp.float32) * di_ref[...]
        o_ref[0] = s[:, :64]
        o_ref[1] = s[:, 64:]
    return pl.pallas_call(
        body,
        grid=(N_NODES // ROW_BLK,),
        in_specs=[
            pl.BlockSpec((ROW_BLK, 128), lambda i: (i, 0)),
            pl.BlockSpec((128, 128), lambda i: (0, 0)),
            pl.BlockSpec((ROW_BLK, 1), lambda i: (i, 0)),
        ],
        out_specs=pl.BlockSpec((2, ROW_BLK, 64), lambda i: (0, i, 0)),
        out_shape=jax.ShapeDtypeStruct((2, N_NODES, 64), jnp.float32),
    )(x, w0p, dinv)


def _tc_mid_fs(p, s2, dinv, bp, wp, fp_out, out_split):
    def body(p_ref, s_ref, di_ref, b_ref, w_ref, o_ref):
        di = di_ref[...]
        pcat = jnp.concatenate([p_ref[0], p_ref[1]], axis=1)
        scat = jnp.concatenate([s_ref[0], s_ref[1]], axis=1)
        h = jnp.maximum(di * pcat + scat * di + b_ref[...], 0.0)
        s_new = jnp.dot(h, w_ref[...],
                        preferred_element_type=jnp.float32) * di
        if out_split:
            o_ref[0] = s_new[:, :64]
            o_ref[1] = s_new[:, 64:]
        else:
            o_ref[...] = s_new
    if out_split:
        out_spec = pl.BlockSpec((2, ROW_BLK, 64), lambda i: (0, i, 0))
        out_shape = jax.ShapeDtypeStruct((2, N_NODES, 64), jnp.float32)
    else:
        out_spec = pl.BlockSpec((ROW_BLK, fp_out), lambda i: (i, 0))
        out_shape = jax.ShapeDtypeStruct((N_NODES, fp_out), jnp.float32)
    return pl.pallas_call(
        body,
        grid=(N_NODES // ROW_BLK,),
        in_specs=[
            pl.BlockSpec((2, ROW_BLK, 64), lambda i: (0, i, 0)),
            pl.BlockSpec((2, ROW_BLK, 64), lambda i: (0, i, 0)),
            pl.BlockSpec((ROW_BLK, 1), lambda i: (i, 0)),
            pl.BlockSpec((1, 128), lambda i: (0, 0)),
            pl.BlockSpec((128, fp_out), lambda i: (0, 0)),
        ],
        out_specs=out_spec,
        out_shape=out_shape,
    )(p, s2, dinv, bp, wp)


def _tc_mid_es(p, s_prev, dinv, bp, wp, fp_in, fp_out):
    def body(p_ref, s_ref, di_ref, b_ref, w_ref, o_ref):
        di = di_ref[...]
        h = jnp.maximum(di * (p_ref[0] + p_ref[1]) + s_ref[...] * di
                        + b_ref[...], 0.0)
        o_ref[...] = jnp.dot(h, w_ref[...],
                             preferred_element_type=jnp.float32) * di
    return pl.pallas_call(
        body,
        grid=(N_NODES // ROW_BLK,),
        in_specs=[
            pl.BlockSpec((2, ROW_BLK, fp_in), lambda i: (0, i, 0)),
            pl.BlockSpec((ROW_BLK, fp_in), lambda i: (i, 0)),
            pl.BlockSpec((ROW_BLK, 1), lambda i: (i, 0)),
            pl.BlockSpec((1, fp_in), lambda i: (0, 0)),
            pl.BlockSpec((fp_in, fp_out), lambda i: (0, 0)),
        ],
        out_specs=pl.BlockSpec((ROW_BLK, fp_out), lambda i: (i, 0)),
        out_shape=jax.ShapeDtypeStruct((N_NODES, fp_out), jnp.float32),
    )(p, s_prev, dinv, bp, wp)


def _tc_last(p, s_prev, dinv, bp):
    def body(p_ref, s_ref, di_ref, b_ref, o_ref):
        di = di_ref[...]
        o_ref[...] = di * (p_ref[0] + p_ref[1]) + s_ref[...] * di + b_ref[...]
    return pl.pallas_call(
        body,
        grid=(N_NODES // ROW_BLK,),
        in_specs=[
            pl.BlockSpec((2, ROW_BLK, 16), lambda i: (0, i, 0)),
            pl.BlockSpec((ROW_BLK, 16), lambda i: (i, 0)),
            pl.BlockSpec((ROW_BLK, 1), lambda i: (i, 0)),
            pl.BlockSpec((1, 16), lambda i: (0, 0)),
        ],
        out_specs=pl.BlockSpec((ROW_BLK, 16), lambda i: (i, 0)),
        out_shape=jax.ShapeDtypeStruct((N_NODES, 16), jnp.float32),
    )(p, s_prev, dinv, bp)


def _pad2(w, r, c):
    return jnp.pad(w, ((0, r - w.shape[0]), (0, c - w.shape[1])))


def kernel(x, edge_index, edge_weight, W0, b0, W1, b1, W2, b2, W3, b3):
    row = edge_index[0].astype(jnp.int32)
    col = edge_index[1].astype(jnp.int32)
    padn = E_PAD - N_EDGES
    rowp = jnp.concatenate([row, jnp.zeros((padn,), jnp.int32)]
                           ).reshape(E_PAD // 128, 128)
    colp = jnp.concatenate([col, jnp.zeros((padn,), jnp.int32)]
                           ).reshape(E_PAD // 128, 128)
    wp = jnp.concatenate([edge_weight, jnp.zeros((padn,), jnp.float32)]
                         ).reshape(E_PAD // 16, 16)

    W0p = _pad2(W0, 128, 128)
    W1p = _pad2(W1, 128, 128)
    W2p = _pad2(W2, 128, 64)
    W3p = _pad2(W3, 64, 16)
    b0p = jnp.pad(b0, (0, 28)).reshape(1, 128)
    b1p = jnp.pad(b1, (0, 28)).reshape(1, 128)
    b2p = jnp.pad(b2, (0, 14)).reshape(1, 64)
    b3p = jnp.pad(b3, (0, 10)).reshape(1, 16)

    ones16 = jnp.ones((N_NODES, 16), jnp.float32)
    degp = _sc_msg16_es(ones16, rowp, colp, wp)
    dinv = _tc_dinv(degp)

    s0 = _tc_first(x, W0p, dinv)
    p0 = _sc_msg64_fs(s0.reshape(2 * N_NODES, 64), rowp, colp, wp)
    s1 = _tc_mid_fs(p0, s0, dinv, b0p, W1p, 128, out_split=True)
    p1 = _sc_msg64_fs(s1.reshape(2 * N_NODES, 64), rowp, colp, wp)
    s2 = _tc_mid_fs(p1, s1, dinv, b1p, W2p, 64, out_split=False)
    p2 = _sc_msg64_es(s2, rowp, colp, wp)
    s3 = _tc_mid_es(p2, s2, dinv, b2p, W3p, 64, 16)
    p3 = _sc_msg16_es(s3, rowp, colp, wp)
    out = _tc_last(p3, s3, dinv, b3p)
    return out[:, :6]


# speedup vs baseline: 9.2630x; 1.0059x over previous
import functools

import jax
import jax.numpy as jnp
from jax import lax
from jax.experimental import pallas as pl
from jax.experimental.pallas import tpu as pltpu
from jax.experimental.pallas import tpu_sc as plsc

N_NODES = 10000
N_EDGES = 320000
NC = 2
NS = 16
NW = NC * NS
E_PAD = 327680
SUPER = 5120
HALF = 256
HALVES = SUPER // HALF
ROW_BLK = 1000
SEG = 624
OUT_SLICE = 208

_GDN = lax.GatherDimensionNumbers(
    offset_dims=(), collapsed_slice_dims=(0,), start_index_map=(0,))


def _bcast_lane(v16, m):
    idx = jnp.full((16, 1), m, jnp.int32)
    return lax.gather(v16, idx, _GDN, (1,),
                      mode=lax.GatherScatterMode.PROMISE_IN_BOUNDS)


def _make_sc_msg(f_pad, fsplit):
    nq = f_pad // 16
    per_tile = E_PAD // NS if fsplit else E_PAD // NW
    n_super = per_tile // SUPER
    mesh = plsc.VectorSubcoreMesh(core_axis_name="c", subcore_axis_name="s")

    @functools.partial(
        pl.kernel,
        out_type=jax.ShapeDtypeStruct((NC, N_NODES, f_pad), jnp.float32),
        mesh=mesh,
        scratch_types=[
            pltpu.VMEM((SUPER // 128, 128), jnp.int32),
            pltpu.VMEM((SUPER // 128, 128), jnp.int32),
            pltpu.VMEM((SUPER // 16, 16), jnp.float32),
            pltpu.VMEM((HALF, f_pad), jnp.float32),
            pltpu.VMEM((HALF, f_pad), jnp.float32),
            pltpu.VMEM_SHARED((N_NODES, f_pad), jnp.float32),
            pltpu.SemaphoreType.DMA,
            pltpu.SemaphoreType.DMA,
            pltpu.SemaphoreType.DMA,
            pltpu.SemaphoreType.DMA,
        ],
        compiler_params=pltpu.CompilerParams(use_tc_tiling_on_sc=False),
    )
    def body(s_hbm, row_hbm, col_hbm, w_hbm, out_hbm,
             idxr, idxc, wv, rows_a, rows_b, acc,
             sem_ga, sem_gb, sem_sa, sem_sb):
        cid = lax.axis_index("c")
        sid = lax.axis_index("s")
        wid = sid if fsplit else sid * NC + cid
        rows = (rows_a, rows_b)
        stage = rows_a.at[pl.ds(0, OUT_SLICE)]
        sem_g = (sem_ga, sem_gb)
        sem_s = (sem_sa, sem_sb)

        def zero_row(i, carry):
            for q in range(nq):
                stage[i, pl.ds(q * 16, 16)] = jnp.zeros((16,), jnp.float32)
            return carry
        lax.fori_loop(0, OUT_SLICE, zero_row, 0)
        for k in range(SEG // OUT_SLICE):
            pltpu.sync_copy(stage,
                            acc.at[pl.ds(sid * SEG + k * OUT_SLICE, OUT_SLICE)])
        @pl.when(sid == NS - 1)
        def _():
            pltpu.sync_copy(stage.at[pl.ds(0, 16)],
                            acc.at[pl.ds(NS * SEG, N_NODES - NS * SEG)])
        plsc.subcore_barrier()

        def issue_gather(h, p):
            for j in range(HALF // 128):
                pltpu.async_copy(s_hbm.at[idxr.at[h * (HALF // 128) + j]],
                                 rows[p].at[pl.ds(j * 128, 128)], sem_g[p])

        def wait_gather(p):
            for j in range(HALF // 128):
                pltpu.make_async_copy(
                    s_hbm.at[idxr.at[j]],
                    rows[p].at[pl.ds(j * 128, 128)], sem_g[p]).wait()

        def issue_scatter(h, p):
            cps = [pltpu.async_copy(rows[p].at[pl.ds(j * 128, 128)],
                                    acc.at[idxc.at[h * (HALF // 128) + j]],
                                    sem_s[p], add=True)
                   for j in range(HALF // 128)]
            for c in cps:
                c.wait()

        def wait_scatter(p):
            pass

        def scale(h, p):
            def _scale_body(l, carry):
                w16 = wv[h * (HALF // 16) + l]
                blk = rows[p].at[pl.ds(l * 16, 16)]
                for m in range(16):
                    wb = _bcast_lane(w16, m)
                    for q in range(nq):
                        blk[m, pl.ds(q * 16, 16)] = (
                            blk[m, pl.ds(q * 16, 16)] * wb)
                return carry
            lax.fori_loop(0, HALF // 16, _scale_body, 0)

        for sc_i in range(n_super):
            base = wid * per_tile + sc_i * SUPER
            pltpu.sync_copy(row_hbm.at[pl.ds(base // 128, SUPER // 128)], idxr)
            pltpu.sync_copy(col_hbm.at[pl.ds(base // 128, SUPER // 128)], idxc)
            pltpu.sync_copy(w_hbm.at[pl.ds(base // 16, SUPER // 16)], wv)
            if fsplit:
                off = jnp.full((16,), cid * N_NODES, jnp.int32)
                def addoff(i, c2):
                    for qq in range(128 // 16):
                        idxr[i, pl.ds(qq * 16, 16)] = (
                            idxr[i, pl.ds(qq * 16, 16)] + off)
                    return c2
                lax.fori_loop(0, SUPER // 128, addoff, 0)

            issue_gather(0, 0)
            def pair(g, carry):
                for hh in range(2):
                    p = hh
                    q = 1 - hh
                    h = g * 2 + hh
                    wait_gather(p)
                    @pl.when(h > 0)
                    def _():
                        wait_scatter(q)
                    @pl.when(h < HALVES - 1)
                    def _():
                        issue_gather(h + 1, q)
                    scale(h, p)
                    issue_scatter(h, p)
                return carry
            lax.fori_loop(0, HALVES // 2, pair, 0)
            wait_scatter(0)
            wait_scatter(1)
        plsc.subcore_barrier()

        for k in range(SEG // OUT_SLICE):
            off = sid * SEG + k * OUT_SLICE
            pltpu.sync_copy(acc.at[pl.ds(off, OUT_SLICE)], stage)
            pltpu.sync_copy(stage, out_hbm.at[cid, pl.ds(off, OUT_SLICE)])
        @pl.when(sid == NS - 1)
        def _():
            tail = N_NODES - NS * SEG
            pltpu.sync_copy(acc.at[pl.ds(NS * SEG, tail)],
                            stage.at[pl.ds(0, tail)])
            pltpu.sync_copy(stage.at[pl.ds(0, tail)],
                            out_hbm.at[cid, pl.ds(NS * SEG, tail)])

    return body


_sc_msg64_fs = _make_sc_msg(64, fsplit=True)
_sc_msg64_es = _make_sc_msg(64, fsplit=False)
_sc_msg16_es = _make_sc_msg(16, fsplit=False)


def _tc_dinv(degp):
    def body(p_ref, o_ref):
        d = p_ref[0, :, 0:1] + p_ref[1, :, 0:1] + 1.0
        o_ref[...] = lax.rsqrt(d)
    return pl.pallas_call(
        body,
        out_shape=jax.ShapeDtypeStruct((N_NODES, 1), jnp.float32),
    )(degp)


def _tc_first(x, w0p, dinv):
    def body(x_ref, w_ref, di_ref, o_ref):
        s = jnp.dot(x_ref[...], w_ref[...],
                    preferred_element_type=jnp.float32) * di_ref[...]
        o_ref[0] = s[:, :64]
        o_ref[1] = s[:, 64:]
    return pl.pallas_call(
        body,
        grid=(N_NODES // ROW_BLK,),
        in_specs=[
            pl.BlockSpec((ROW_BLK, 128), lambda i: (i, 0)),
            pl.BlockSpec((128, 128), lambda i: (0, 0)),
            pl.BlockSpec((ROW_BLK, 1), lambda i: (i, 0)),
        ],
        out_specs=pl.BlockSpec((2, ROW_BLK, 64), lambda i: (0, i, 0)),
        out_shape=jax.ShapeDtypeStruct((2, N_NODES, 64), jnp.float32),
    )(x, w0p, dinv)


def _tc_mid_fs(p, s2, dinv, bp, wp, fp_out, out_split):
    def body(p_ref, s_ref, di_ref, b_ref, w_ref, o_ref):
        di = di_ref[...]
        pcat = jnp.concatenate([p_ref[0], p_ref[1]], axis=1)
        scat = jnp.concatenate([s_ref[0], s_ref[1]], axis=1)
        h = jnp.maximum(di * pcat + scat * di + b_ref[...], 0.0)
        s_new = jnp.dot(h, w_ref[...],
                        preferred_element_type=jnp.float32) * di
        if out_split:
            o_ref[0] = s_new[:, :64]
            o_ref[1] = s_new[:, 64:]
        else:
            o_ref[...] = s_new
    if out_split:
        out_spec = pl.BlockSpec((2, ROW_BLK, 64), lambda i: (0, i, 0))
        out_shape = jax.ShapeDtypeStruct((2, N_NODES, 64), jnp.float32)
    else:
        out_spec = pl.BlockSpec((ROW_BLK, fp_out), lambda i: (i, 0))
        out_shape = jax.ShapeDtypeStruct((N_NODES, fp_out), jnp.float32)
    return pl.pallas_call(
        body,
        grid=(N_NODES // ROW_BLK,),
        in_specs=[
            pl.BlockSpec((2, ROW_BLK, 64), lambda i: (0, i, 0)),
            pl.BlockSpec((2, ROW_BLK, 64), lambda i: (0, i, 0)),
            pl.BlockSpec((ROW_BLK, 1), lambda i: (i, 0)),
            pl.BlockSpec((1, 128), lambda i: (0, 0)),
            pl.BlockSpec((128, fp_out), lambda i: (0, 0)),
        ],
        out_specs=out_spec,
        out_shape=out_shape,
    )(p, s2, dinv, bp, wp)


def _tc_mid_es(p, s_prev, dinv, bp, wp, fp_in, fp_out):
    def body(p_ref, s_ref, di_ref, b_ref, w_ref, o_ref):
        di = di_ref[...]
        h = jnp.maximum(di * (p_ref[0] + p_ref[1]) + s_ref[...] * di
                        + b_ref[...], 0.0)
        o_ref[...] = jnp.dot(h, w_ref[...],
                             preferred_element_type=jnp.float32) * di
    return pl.pallas_call(
        body,
        grid=(N_NODES // ROW_BLK,),
        in_specs=[
            pl.BlockSpec((2, ROW_BLK, fp_in), lambda i: (0, i, 0)),
            pl.BlockSpec((ROW_BLK, fp_in), lambda i: (i, 0)),
            pl.BlockSpec((ROW_BLK, 1), lambda i: (i, 0)),
            pl.BlockSpec((1, fp_in), lambda i: (0, 0)),
            pl.BlockSpec((fp_in, fp_out), lambda i: (0, 0)),
        ],
        out_specs=pl.BlockSpec((ROW_BLK, fp_out), lambda i: (i, 0)),
        out_shape=jax.ShapeDtypeStruct((N_NODES, fp_out), jnp.float32),
    )(p, s_prev, dinv, bp, wp)


def _tc_last(p, s_prev, dinv, bp):
    def body(p_ref, s_ref, di_ref, b_ref, o_ref):
        di = di_ref[...]
        o_ref[...] = di * (p_ref[0] + p_ref[1]) + s_ref[...] * di + b_ref[...]
    return pl.pallas_call(
        body,
        grid=(N_NODES // ROW_BLK,),
        in_specs=[
            pl.BlockSpec((2, ROW_BLK, 16), lambda i: (0, i, 0)),
            pl.BlockSpec((ROW_BLK, 16), lambda i: (i, 0)),
            pl.BlockSpec((ROW_BLK, 1), lambda i: (i, 0)),
            pl.BlockSpec((1, 16), lambda i: (0, 0)),
        ],
        out_specs=pl.BlockSpec((ROW_BLK, 16), lambda i: (i, 0)),
        out_shape=jax.ShapeDtypeStruct((N_NODES, 16), jnp.float32),
    )(p, s_prev, dinv, bp)


def _pad2(w, r, c):
    return jnp.pad(w, ((0, r - w.shape[0]), (0, c - w.shape[1])))


def kernel(x, edge_index, edge_weight, W0, b0, W1, b1, W2, b2, W3, b3):
    row = edge_index[0].astype(jnp.int32)
    col = edge_index[1].astype(jnp.int32)
    padn = E_PAD - N_EDGES
    rowp = jnp.concatenate([row, jnp.zeros((padn,), jnp.int32)]
                           ).reshape(E_PAD // 128, 128)
    colp = jnp.concatenate([col, jnp.zeros((padn,), jnp.int32)]
                           ).reshape(E_PAD // 128, 128)
    wp = jnp.concatenate([edge_weight, jnp.zeros((padn,), jnp.float32)]
                         ).reshape(E_PAD // 16, 16)

    W0p = _pad2(W0, 128, 128)
    W1p = _pad2(W1, 128, 128)
    W2p = _pad2(W2, 128, 64)
    W3p = _pad2(W3, 64, 16)
    b0p = jnp.pad(b0, (0, 28)).reshape(1, 128)
    b1p = jnp.pad(b1, (0, 28)).reshape(1, 128)
    b2p = jnp.pad(b2, (0, 14)).reshape(1, 64)
    b3p = jnp.pad(b3, (0, 10)).reshape(1, 16)

    ones16 = jnp.ones((N_NODES, 16), jnp.float32)
    degp = _sc_msg16_es(ones16, rowp, colp, wp)
    dinv = _tc_dinv(degp)

    s0 = _tc_first(x, W0p, dinv)
    p0 = _sc_msg64_fs(s0.reshape(2 * N_NODES, 64), rowp, colp, wp)
    s1 = _tc_mid_fs(p0, s0, dinv, b0p, W1p, 128, out_split=True)
    p1 = _sc_msg64_fs(s1.reshape(2 * N_NODES, 64), rowp, colp, wp)
    s2 = _tc_mid_fs(p1, s1, dinv, b1p, W2p, 64, out_split=False)
    p2 = _sc_msg64_es(s2, rowp, colp, wp)
    s3 = _tc_mid_es(p2, s2, dinv, b2p, W3p, 64, 16)
    p3 = _sc_msg16_es(s3, rowp, colp, wp)
    out = _tc_last(p3, s3, dinv, b3p)
    return out[:, :6]


# speedup vs baseline: 9.7154x; 1.0488x over previous
import functools

import jax
import jax.numpy as jnp
from jax import lax
from jax.experimental import pallas as pl
from jax.experimental.pallas import tpu as pltpu
from jax.experimental.pallas import tpu_sc as plsc

N_NODES = 10000
N_EDGES = 320000
NC = 2
NS = 16
NW = NC * NS
E_PAD = 327680
SUPER = 5120
HALF = 128
HALVES = SUPER // HALF
NBUF = 4
ROW_BLK = 1000
SEG = 624
OUT_CHUNK = 48

_GDN = lax.GatherDimensionNumbers(
    offset_dims=(), collapsed_slice_dims=(0,), start_index_map=(0,))


def _bcast_lane(v16, m):
    idx = jnp.full((16, 1), m, jnp.int32)
    return lax.gather(v16, idx, _GDN, (1,),
                      mode=lax.GatherScatterMode.PROMISE_IN_BOUNDS)


def _make_sc_msg(f_pad, fsplit):
    nq = f_pad // 16
    per_tile = E_PAD // NS if fsplit else E_PAD // NW
    n_super = per_tile // SUPER
    mesh = plsc.VectorSubcoreMesh(core_axis_name="c", subcore_axis_name="s")

    @functools.partial(
        pl.kernel,
        out_type=jax.ShapeDtypeStruct((NC, N_NODES, f_pad), jnp.float32),
        mesh=mesh,
        scratch_types=[
            pltpu.VMEM((SUPER // 128, 128), jnp.int32),
            pltpu.VMEM((SUPER // 128, 128), jnp.int32),
            pltpu.VMEM((SUPER // 16, 16), jnp.float32),
        ] + [pltpu.VMEM((HALF, f_pad), jnp.float32) for _ in range(NBUF)]
        + [
            pltpu.VMEM_SHARED((N_NODES, f_pad), jnp.float32),
        ] + [pltpu.SemaphoreType.DMA for _ in range(NBUF)]
        + [pltpu.SemaphoreType.DMA],
        compiler_params=pltpu.CompilerParams(use_tc_tiling_on_sc=False),
    )
    def body(s_hbm, row_hbm, col_hbm, w_hbm, out_hbm, idxr, idxc, wv, *rest):
        rows = rest[:NBUF]
        acc = rest[NBUF]
        sem_g = rest[NBUF + 1:2 * NBUF + 1]
        sem_s = rest[2 * NBUF + 1]
        cid = lax.axis_index("c")
        sid = lax.axis_index("s")
        wid = sid if fsplit else sid * NC + cid
        stage = rows[0].at[pl.ds(0, OUT_CHUNK)]

        def zero_row(i, carry):
            for q in range(nq):
                stage[i, pl.ds(q * 16, 16)] = jnp.zeros((16,), jnp.float32)
            return carry
        lax.fori_loop(0, OUT_CHUNK, zero_row, 0)
        for k in range(SEG // OUT_CHUNK):
            pltpu.sync_copy(stage,
                            acc.at[pl.ds(sid * SEG + k * OUT_CHUNK, OUT_CHUNK)])
        @pl.when(sid == NS - 1)
        def _():
            pltpu.sync_copy(stage.at[pl.ds(0, 16)],
                            acc.at[pl.ds(NS * SEG, N_NODES - NS * SEG)])
        plsc.subcore_barrier()

        def issue_gather(h, p):
            pltpu.async_copy(s_hbm.at[idxr.at[h]], rows[p], sem_g[p])

        def wait_gather(p):
            pltpu.make_async_copy(s_hbm.at[idxr.at[0]], rows[p],
                                  sem_g[p]).wait()

        def scale_and_scatter(h, p):
            def _scale_body(l, carry):
                w16 = wv[h * (HALF // 16) + l]
                blk = rows[p].at[pl.ds(l * 16, 16)]
                for m in range(16):
                    wb = _bcast_lane(w16, m)
                    for q in range(nq):
                        blk[m, pl.ds(q * 16, 16)] = (
                            blk[m, pl.ds(q * 16, 16)] * wb)
                return carry
            lax.fori_loop(0, HALF // 16, _scale_body, 0)
            pltpu.async_copy(rows[p], acc.at[idxc.at[h]], sem_s,
                             add=True).wait()

        def super_chunk(sc_i, carry):
            base = wid * per_tile + sc_i * SUPER
            b128 = pl.multiple_of(base // 128, 8)
            b16 = pl.multiple_of(base // 16, 8)
            pltpu.sync_copy(row_hbm.at[pl.ds(b128, SUPER // 128)], idxr)
            pltpu.sync_copy(col_hbm.at[pl.ds(b128, SUPER // 128)], idxc)
            pltpu.sync_copy(w_hbm.at[pl.ds(b16, SUPER // 16)], wv)
            if fsplit:
                off = jnp.full((16,), cid * N_NODES, jnp.int32)
                def addoff(i, c2):
                    for qq in range(128 // 16):
                        idxr[i, pl.ds(qq * 16, 16)] = (
                            idxr[i, pl.ds(qq * 16, 16)] + off)
                    return c2
                lax.fori_loop(0, SUPER // 128, addoff, 0)

            for hh in range(NBUF - 1):
                issue_gather(hh, hh)
            def ring(g, c2):
                for hh in range(NBUF):
                    p = hh
                    h = g * NBUF + hh
                    wait_gather(p)
                    @pl.when(h + NBUF - 1 < HALVES)
                    def _():
                        issue_gather(h + NBUF - 1, (p + NBUF - 1) % NBUF)
                    scale_and_scatter(h, p)
                return c2
            lax.fori_loop(0, HALVES // NBUF, ring, 0)
            return carry
        lax.fori_loop(0, n_super, super_chunk, 0)
        plsc.subcore_barrier()

        for k in range(SEG // OUT_CHUNK):
            off = sid * SEG + k * OUT_CHUNK
            pltpu.sync_copy(acc.at[pl.ds(off, OUT_CHUNK)], stage)
            pltpu.sync_copy(stage, out_hbm.at[cid, pl.ds(off, OUT_CHUNK)])
        @pl.when(sid == NS - 1)
        def _():
            tail = N_NODES - NS * SEG
            pltpu.sync_copy(acc.at[pl.ds(NS * SEG, tail)],
                            stage.at[pl.ds(0, tail)])
            pltpu.sync_copy(stage.at[pl.ds(0, tail)],
                            out_hbm.at[cid, pl.ds(NS * SEG, tail)])

    return body


_sc_msg64_fs = _make_sc_msg(64, fsplit=True)
_sc_msg64_es = _make_sc_msg(64, fsplit=False)
_sc_msg16_es = _make_sc_msg(16, fsplit=False)


def _tc_dinv(degp):
    def body(p_ref, o_ref):
        d = p_ref[0, :, 0:1] + p_ref[1, :, 0:1] + 1.0
        o_ref[...] = lax.rsqrt(d)
    return pl.pallas_call(
        body,
        out_shape=jax.ShapeDtypeStruct((N_NODES, 1), jnp.float32),
    )(degp)


def _tc_first(x, w0p, dinv):
    def body(x_ref, w_ref, di_ref, o_ref):
        s = jnp.dot(x_ref[...], w_ref[...],
                    preferred_element_type=jnp.float32) * di_ref[...]
        o_ref[0] = s[:, :64]
        o_ref[1] = s[:, 64:]
    return pl.pallas_call(
        body,
        grid=(N_NODES // ROW_BLK,),
        in_specs=[
            pl.BlockSpec((ROW_BLK, 128), lambda i: (i, 0)),
            pl.BlockSpec((128, 128), lambda i: (0, 0)),
            pl.BlockSpec((ROW_BLK, 1), lambda i: (i, 0)),
        ],
        out_specs=pl.BlockSpec((2, ROW_BLK, 64), lambda i: (0, i, 0)),
        out_shape=jax.ShapeDtypeStruct((2, N_NODES, 64), jnp.float32),
    )(x, w0p, dinv)


def _tc_mid_fs(p, s2, dinv, bp, wp, fp_out, out_split):
    def body(p_ref, s_ref, di_ref, b_ref, w_ref, o_ref):
        di = di_ref[...]
        pcat = jnp.concatenate([p_ref[0], p_ref[1]], axis=1)
        scat = jnp.concatenate([s_ref[0], s_ref[1]], axis=1)
        h = jnp.maximum(di * pcat + scat * di + b_ref[...], 0.0)
        s_new = jnp.dot(h, w_ref[...],
                        preferred_element_type=jnp.float32) * di
        if out_split:
            o_ref[0] = s_new[:, :64]
            o_ref[1] = s_new[:, 64:]
        else:
            o_ref[...] = s_new
    if out_split:
        out_spec = pl.BlockSpec((2, ROW_BLK, 64), lambda i: (0, i, 0))
        out_shape = jax.ShapeDtypeStruct((2, N_NODES, 64), jnp.float32)
    else:
        out_spec = pl.BlockSpec((ROW_BLK, fp_out), lambda i: (i, 0))
        out_shape = jax.ShapeDtypeStruct((N_NODES, fp_out), jnp.float32)
    return pl.pallas_call(
        body,
        grid=(N_NODES // ROW_BLK,),
        in_specs=[
            pl.BlockSpec((2, ROW_BLK, 64), lambda i: (0, i, 0)),
            pl.BlockSpec((2, ROW_BLK, 64), lambda i: (0, i, 0)),
            pl.BlockSpec((ROW_BLK, 1), lambda i: (i, 0)),
            pl.BlockSpec((1, 128), lambda i: (0, 0)),
            pl.BlockSpec((128, fp_out), lambda i: (0, 0)),
        ],
        out_specs=out_spec,
        out_shape=out_shape,
    )(p, s2, dinv, bp, wp)


def _tc_mid_es(p, s_prev, dinv, bp, wp, fp_in, fp_out):
    def body(p_ref, s_ref, di_ref, b_ref, w_ref, o_ref):
        di = di_ref[...]
        h = jnp.maximum(di * (p_ref[0] + p_ref[1]) + s_ref[...] * di
                        + b_ref[...], 0.0)
        o_ref[...] = jnp.dot(h, w_ref[...],
                             preferred_element_type=jnp.float32) * di
    return pl.pallas_call(
        body,
        grid=(N_NODES // ROW_BLK,),
        in_specs=[
            pl.BlockSpec((2, ROW_BLK, fp_in), lambda i: (0, i, 0)),
            pl.BlockSpec((ROW_BLK, fp_in), lambda i: (i, 0)),
            pl.BlockSpec((ROW_BLK, 1), lambda i: (i, 0)),
            pl.BlockSpec((1, fp_in), lambda i: (0, 0)),
            pl.BlockSpec((fp_in, fp_out), lambda i: (0, 0)),
        ],
        out_specs=pl.BlockSpec((ROW_BLK, fp_out), lambda i: (i, 0)),
        out_shape=jax.ShapeDtypeStruct((N_NODES, fp_out), jnp.float32),
    )(p, s_prev, dinv, bp, wp)


def _tc_last(p, s_prev, dinv, bp):
    def body(p_ref, s_ref, di_ref, b_ref, o_ref):
        di = di_ref[...]
        o_ref[...] = di * (p_ref[0] + p_ref[1]) + s_ref[...] * di + b_ref[...]
    return pl.pallas_call(
        body,
        grid=(N_NODES // ROW_BLK,),
        in_specs=[
            pl.BlockSpec((2, ROW_BLK, 16), lambda i: (0, i, 0)),
            pl.BlockSpec((ROW_BLK, 16), lambda i: (i, 0)),
            pl.BlockSpec((ROW_BLK, 1), lambda i: (i, 0)),
            pl.BlockSpec((1, 16), lambda i: (0, 0)),
        ],
        out_specs=pl.BlockSpec((ROW_BLK, 16), lambda i: (i, 0)),
        out_shape=jax.ShapeDtypeStruct((N_NODES, 16), jnp.float32),
    )(p, s_prev, dinv, bp)


def _pad2(w, r, c):
    return jnp.pad(w, ((0, r - w.shape[0]), (0, c - w.shape[1])))


def kernel(x, edge_index, edge_weight, W0, b0, W1, b1, W2, b2, W3, b3):
    row = edge_index[0].astype(jnp.int32)
    col = edge_index[1].astype(jnp.int32)
    padn = E_PAD - N_EDGES
    rowp = jnp.concatenate([row, jnp.zeros((padn,), jnp.int32)]
                           ).reshape(E_PAD // 128, 128)
    colp = jnp.concatenate([col, jnp.zeros((padn,), jnp.int32)]
                           ).reshape(E_PAD // 128, 128)
    wp = jnp.concatenate([edge_weight, jnp.zeros((padn,), jnp.float32)]
                         ).reshape(E_PAD // 16, 16)

    W0p = _pad2(W0, 128, 128)
    W1p = _pad2(W1, 128, 128)
    W2p = _pad2(W2, 128, 64)
    W3p = _pad2(W3, 64, 16)
    b0p = jnp.pad(b0, (0, 28)).reshape(1, 128)
    b1p = jnp.pad(b1, (0, 28)).reshape(1, 128)
    b2p = jnp.pad(b2, (0, 14)).reshape(1, 64)
    b3p = jnp.pad(b3, (0, 10)).reshape(1, 16)

    ones16 = jnp.ones((N_NODES, 16), jnp.float32)
    degp = _sc_msg16_es(ones16, rowp, colp, wp)
    dinv = _tc_dinv(degp)

    s0 = _tc_first(x, W0p, dinv)
    p0 = _sc_msg64_fs(s0.reshape(2 * N_NODES, 64), rowp, colp, wp)
    s1 = _tc_mid_fs(p0, s0, dinv, b0p, W1p, 128, out_split=True)
    p1 = _sc_msg64_fs(s1.reshape(2 * N_NODES, 64), rowp, colp, wp)
    s2 = _tc_mid_fs(p1, s1, dinv, b1p, W2p, 64, out_split=False)
    p2 = _sc_msg64_es(s2, rowp, colp, wp)
    s3 = _tc_mid_es(p2, s2, dinv, b2p, W3p, 64, 16)
    p3 = _sc_msg16_es(s3, rowp, colp, wp)
    out = _tc_last(p3, s3, dinv, b3p)
    return out[:, :6]
